# baseline (speedup 1.0000x reference)
import jax
import jax.numpy as jnp
from jax import lax
from jax.experimental import pallas as pl
from jax.experimental.pallas import tpu as pltpu
from jax.experimental.pallas import tpu_sc as plsc

_NC = 2
_NS = 16
_L = 16
_CH = 128
_CH2 = 64


def _sc_degree(col, ew, n_pad, n_chunks):
    seg = n_pad // _NS
    mesh = plsc.VectorSubcoreMesh(core_axis_name="c", subcore_axis_name="s")

    def body(col_hbm, ew_hbm, out_hbm, colbuf, ewbuf, vbuf, deg_sh, sem):
        c = lax.axis_index("c")
        s = lax.axis_index("s")
        wid = c * _NS + s

        def zero(i, carry):
            vbuf[pl.ds(i * _L, _L)] = jnp.zeros((_L,), jnp.float32)
            return carry

        lax.fori_loop(0, seg // _L, zero, 0)
        pltpu.sync_copy(vbuf, deg_sh.at[pl.ds(s * seg, seg)])
        pltpu.sync_copy(col_hbm.at[pl.ds(wid * n_chunks, n_chunks)], colbuf)
        pltpu.sync_copy(ew_hbm.at[pl.ds(wid * n_chunks, n_chunks)], ewbuf)
        plsc.subcore_barrier()

        def chunk(j, carry):
            pltpu.async_copy(ewbuf.at[j], deg_sh.at[colbuf.at[j]], sem,
                             add=True)
            return carry

        lax.fori_loop(0, n_chunks, chunk, 0)
        pltpu.make_async_copy(ew_hbm.at[pl.ds(0, n_chunks)], ewbuf, sem).wait()
        plsc.subcore_barrier()
        pltpu.sync_copy(deg_sh.at[pl.ds(s * seg, seg)], vbuf)
        pltpu.sync_copy(vbuf, out_hbm.at[pl.ds(c * n_pad + s * seg, seg)])

    run = pl.kernel(
        body,
        out_type=jax.ShapeDtypeStruct((_NC * n_pad,), jnp.float32),
        mesh=mesh,
        scratch_types=[
            pltpu.VMEM((n_chunks, _CH), jnp.int32),
            pltpu.VMEM((n_chunks, _CH), jnp.float32),
            pltpu.VMEM((n_pad // _NS,), jnp.float32),
            pltpu.VMEM_SHARED((n_pad,), jnp.float32),
            pltpu.SemaphoreType.DMA,
        ],
    )
    return run(col, ew)


def _sc_message_scatter(row, col, ew, y, n_pad, n_chunks, d):
    seg = n_pad // _NS
    quarter = n_chunks // 4
    quads = quarter // 4
    n_dump = seg // _CH2
    mesh = plsc.VectorSubcoreMesh(core_axis_name="c", subcore_axis_name="s")

    def body(row_hbm, col_hbm, ew_hbm, y_hbm, out_hbm,
             rowbuf, colbuf, ewbuf, r0, r1, r2, r3, acc_sh, s0, s1, s2, s3):
        c = lax.axis_index("c")
        s = lax.axis_index("s")
        wid = c * _NS + s
        bufs = ((r0, s0), (r1, s1), (r2, s2), (r3, s3))

        def zrow(r, carry):
            for f in range(d // _L):
                r0[r, pl.ds(f * _L, _L)] = jnp.zeros((_L,), jnp.float32)
            return carry

        lax.fori_loop(0, _CH2, zrow, 0)

        def zcopy(i, carry):
            pltpu.sync_copy(r0, acc_sh.at[pl.ds(s * seg + i * _CH2, _CH2)])
            return carry

        lax.fori_loop(0, n_dump, zcopy, 0)
        plsc.subcore_barrier()

        dummy = y_hbm.at[pl.ds(0, _CH2)]

        dn = lax.GatherDimensionNumbers(
            offset_dims=(), collapsed_slice_dims=(0,), start_index_map=(0,))

        def scale(rows, j):
            def grp(g, cc):
                ew16 = ewbuf[j, pl.ds(g * _L, _L)]
                for e in range(_L):
                    eidx = g * _L + e
                    idxv = jnp.full((_L, 1), e, dtype=jnp.int32)
                    spl = lax.gather(ew16, idxv, dn, slice_sizes=(1,),
                                     mode=lax.GatherScatterMode.PROMISE_IN_BOUNDS)
                    for f in range(d // _L):
                        sl = pl.ds(f * _L, _L)
                        rows[eidx, sl] = rows[eidx, sl] * spl
                return cc

            lax.fori_loop(0, _CH2 // _L, grp, 0)

        def quad(q, carry):
            for b in range(4):
                rows, sem = bufs[b]
                j = 4 * q + b
                pltpu.make_async_copy(dummy, rows, sem).wait()
                scale(rows, j)
                pltpu.sync_copy(rows, acc_sh.at[colbuf.at[j]], add=True)

                @pl.when(j + 4 < quarter)
                def _():
                    pltpu.async_copy(y_hbm.at[rowbuf.at[j + 4]], rows, sem)

            return carry

        for p in range(4):
            base_chunk = wid * n_chunks + p * quarter
            pltpu.sync_copy(row_hbm.at[pl.ds(base_chunk, quarter)], rowbuf)
            pltpu.sync_copy(col_hbm.at[pl.ds(base_chunk, quarter)], colbuf)
            pltpu.sync_copy(ew_hbm.at[pl.ds(base_chunk, quarter)], ewbuf)
            for b in range(4):
                rows, sem = bufs[b]
                pltpu.async_copy(y_hbm.at[rowbuf.at[b]], rows, sem)
            lax.fori_loop(0, quads, quad, 0)

        plsc.subcore_barrier()

        for i in range(n_dump):
            buf, sem = bufs[i % 2]
            if i >= 2:
                pltpu.make_async_copy(dummy, buf, sem).wait()
            base = s * seg + i * _CH2
            pltpu.sync_copy(acc_sh.at[pl.ds(base, _CH2)], buf)
            pltpu.async_copy(buf, out_hbm.at[pl.ds(c * n_pad + base, _CH2)],
                             sem)
        for i in range(max(0, n_dump - 2), n_dump):
            buf, sem = bufs[i % 2]
            pltpu.make_async_copy(dummy, buf, sem).wait()

    run = pl.kernel(
        body,
        out_type=jax.ShapeDtypeStruct((_NC * n_pad, d), jnp.float32),
        mesh=mesh,
        scratch_types=[
            pltpu.VMEM((quarter, _CH2), jnp.int32),
            pltpu.VMEM((quarter, _CH2), jnp.int32),
            pltpu.VMEM((quarter, _CH2), jnp.float32),
            pltpu.VMEM((_CH2, d), jnp.float32),
            pltpu.VMEM((_CH2, d), jnp.float32),
            pltpu.VMEM((_CH2, d), jnp.float32),
            pltpu.VMEM((_CH2, d), jnp.float32),
            pltpu.VMEM_SHARED((n_pad, d), jnp.float32),
            pltpu.SemaphoreType.DMA,
            pltpu.SemaphoreType.DMA,
            pltpu.SemaphoreType.DMA,
            pltpu.SemaphoreType.DMA,
        ],
    )
    return run(row, col, ew, y)


def _tc_norm_y(d0, d1, x, W, rb):
    n, d = x.shape

    def body(d0_ref, d1_ref, x_ref, w_ref, dis_ref, y_ref):
        deg = 1.0 + d0_ref[...] + d1_ref[...]
        dis = lax.rsqrt(deg)
        xw = lax.dot_general(x_ref[...], w_ref[...],
                             (((1,), (1,)), ((), ())),
                             preferred_element_type=jnp.float32)
        dis_ref[...] = dis
        y_ref[...] = dis * xw

    return pl.pallas_call(
        body,
        grid=(n // rb,),
        in_specs=[
            pl.BlockSpec((rb, 1), lambda i: (i, 0)),
            pl.BlockSpec((rb, 1), lambda i: (i, 0)),
            pl.BlockSpec((rb, d), lambda i: (i, 0)),
            pl.BlockSpec((d, d), lambda i: (0, 0)),
        ],
        out_specs=[
            pl.BlockSpec((rb, 1), lambda i: (i, 0)),
            pl.BlockSpec((rb, d), lambda i: (i, 0)),
        ],
        out_shape=[
            jax.ShapeDtypeStruct((n, 1), jnp.float32),
            jax.ShapeDtypeStruct((n, d), jnp.float32),
        ],
    )(d0, d1, x, W)


def _tc_combine(dis, y, p0, p1, rb):
    n, d = y.shape

    def body(dis_ref, y_ref, p0_ref, p1_ref, o_ref):
        o_ref[...] = dis_ref[...] * (p0_ref[...] + p1_ref[...] + y_ref[...])

    return pl.pallas_call(
        body,
        grid=(n // rb,),
        in_specs=[
            pl.BlockSpec((rb, 1), lambda i: (i, 0)),
            pl.BlockSpec((rb, d), lambda i: (i, 0)),
            pl.BlockSpec((rb, d), lambda i: (i, 0)),
            pl.BlockSpec((rb, d), lambda i: (i, 0)),
        ],
        out_specs=pl.BlockSpec((rb, d), lambda i: (i, 0)),
        out_shape=jax.ShapeDtypeStruct((n, d), jnp.float32),
    )(dis, y, p0, p1)


@jax.jit
def kernel(x, edge_index, edge_weight, W):
    n, d = x.shape
    e = edge_weight.shape[0]
    ntile = _NC * _NS

    quantum = ntile * _CH2 * 16
    ep = ((e + quantum - 1) // quantum) * quantum
    n_chunks = ep // (ntile * _CH)
    pad = ep - e
    row = jnp.concatenate([edge_index[0], jnp.zeros((pad,), jnp.int32)])
    col = jnp.concatenate([edge_index[1], jnp.zeros((pad,), jnp.int32)])
    ewp = jnp.concatenate([edge_weight, jnp.zeros((pad,), jnp.float32)])
    row2 = row.reshape(ntile * n_chunks, _CH)
    col2 = col.reshape(ntile * n_chunks, _CH)
    ew2 = ewp.reshape(ntile * n_chunks, _CH)
    n_chunks2 = ep // (ntile * _CH2)
    row3 = row.reshape(ntile * n_chunks2, _CH2)
    col3 = col.reshape(ntile * n_chunks2, _CH2)
    ew3 = ewp.reshape(ntile * n_chunks2, _CH2)

    n_pad = ((n + _NS * _CH - 1) // (_NS * _CH)) * (_NS * _CH)

    dp = _sc_degree(col2, ew2, n_pad, n_chunks).reshape(_NC, n_pad)
    d0 = dp[0, :n, None]
    d1 = dp[1, :n, None]

    rb = 1000 if n % 1000 == 0 else 8
    dis, y = _tc_norm_y(d0, d1, x, W, rb)

    partials = _sc_message_scatter(row3, col3, ew3, y, n_pad,
                                   n_chunks2, d).reshape(_NC, n_pad, d)
    p0 = partials[0, :n]
    p1 = partials[1, :n]

    return _tc_combine(dis, y, p0, p1, rb)


# speedup vs baseline: 1.0141x; 1.0141x over previous
import jax
import jax.numpy as jnp
from jax import lax
from jax.experimental import pallas as pl
from jax.experimental.pallas import tpu as pltpu
from jax.experimental.pallas import tpu_sc as plsc

_NC = 2
_NS = 16
_L = 16
_CH = 128


def _sc_degree(col, ew, n_pad, n_chunks):
    seg = n_pad // _NS
    mesh = plsc.VectorSubcoreMesh(core_axis_name="c", subcore_axis_name="s")

    def body(col_hbm, ew_hbm, out_hbm, colbuf, ewbuf, vbuf, deg_sh, sem):
        c = lax.axis_index("c")
        s = lax.axis_index("s")
        wid = c * _NS + s

        def zero(i, carry):
            vbuf[pl.ds(i * _L, _L)] = jnp.zeros((_L,), jnp.float32)
            return carry

        lax.fori_loop(0, seg // _L, zero, 0)
        pltpu.sync_copy(vbuf, deg_sh.at[pl.ds(s * seg, seg)])
        pltpu.sync_copy(col_hbm.at[pl.ds(wid * n_chunks, n_chunks)], colbuf)
        pltpu.sync_copy(ew_hbm.at[pl.ds(wid * n_chunks, n_chunks)], ewbuf)
        plsc.subcore_barrier()

        def chunk(j, carry):
            pltpu.async_copy(ewbuf.at[j], deg_sh.at[colbuf.at[j]], sem,
                             add=True)
            return carry

        lax.fori_loop(0, n_chunks, chunk, 0)
        pltpu.make_async_copy(ew_hbm.at[pl.ds(0, n_chunks)], ewbuf, sem).wait()
        plsc.subcore_barrier()
        pltpu.sync_copy(deg_sh.at[pl.ds(s * seg, seg)], vbuf)
        pltpu.sync_copy(vbuf, out_hbm.at[pl.ds(c * n_pad + s * seg, seg)])

    run = pl.kernel(
        body,
        out_type=jax.ShapeDtypeStruct((_NC * n_pad,), jnp.float32),
        mesh=mesh,
        scratch_types=[
            pltpu.VMEM((n_chunks, _CH), jnp.int32),
            pltpu.VMEM((n_chunks, _CH), jnp.float32),
            pltpu.VMEM((n_pad // _NS,), jnp.float32),
            pltpu.VMEM_SHARED((n_pad,), jnp.float32),
            pltpu.SemaphoreType.DMA,
        ],
    )
    return run(col, ew)


def _sc_message_scatter(row, col, ew, y, n_pad, n_chunks, d):
    seg = n_pad // _NS
    half = n_chunks // 2
    hpairs = half // 2
    n_dump = seg // _CH
    mesh = plsc.VectorSubcoreMesh(core_axis_name="c", subcore_axis_name="s")

    def body(row_hbm, col_hbm, ew_hbm, y_hbm, out_hbm,
             rowbuf, colbuf, ewbuf, rows0, rows1, acc_sh, sem0, sem1):
        c = lax.axis_index("c")
        s = lax.axis_index("s")
        wid = c * _NS + s

        def zrow(r, carry):
            for f in range(d // _L):
                rows0[r, pl.ds(f * _L, _L)] = jnp.zeros((_L,), jnp.float32)
            return carry

        lax.fori_loop(0, _CH, zrow, 0)

        def zcopy(i, carry):
            pltpu.sync_copy(rows0, acc_sh.at[pl.ds(s * seg + i * _CH, _CH)])
            return carry

        lax.fori_loop(0, n_dump, zcopy, 0)
        plsc.subcore_barrier()

        dummy = y_hbm.at[pl.ds(0, _CH)]

        dn = lax.GatherDimensionNumbers(
            offset_dims=(), collapsed_slice_dims=(0,), start_index_map=(0,))

        def scale(rows, j):
            def grp(g, cc):
                ew16 = ewbuf[j, pl.ds(g * _L, _L)]
                for e in range(_L):
                    eidx = g * _L + e
                    idxv = jnp.full((_L, 1), e, dtype=jnp.int32)
                    spl = lax.gather(ew16, idxv, dn, slice_sizes=(1,),
                                     mode=lax.GatherScatterMode.PROMISE_IN_BOUNDS)
                    for f in range(d // _L):
                        sl = pl.ds(f * _L, _L)
                        rows[eidx, sl] = rows[eidx, sl] * spl
                return cc

            lax.fori_loop(0, _CH // _L, grp, 0)

        def pair(j2, carry):
            j0 = 2 * j2
            pltpu.make_async_copy(dummy, rows0, sem0).wait()
            pltpu.async_copy(y_hbm.at[rowbuf.at[j0 + 1]], rows1, sem1)
            scale(rows0, j0)
            pltpu.sync_copy(rows0, acc_sh.at[colbuf.at[j0]], add=True)

            pltpu.make_async_copy(dummy, rows1, sem1).wait()

            @pl.when(j2 + 1 < hpairs)
            def _():
                pltpu.async_copy(y_hbm.at[rowbuf.at[j0 + 2]], rows0, sem0)

            scale(rows1, j0 + 1)
            pltpu.sync_copy(rows1, acc_sh.at[colbuf.at[j0 + 1]], add=True)
            return carry

        for p in range(2):
            base_chunk = wid * n_chunks + p * half
            pltpu.sync_copy(row_hbm.at[pl.ds(base_chunk, half)], rowbuf)
            pltpu.sync_copy(col_hbm.at[pl.ds(base_chunk, half)], colbuf)
            pltpu.sync_copy(ew_hbm.at[pl.ds(base_chunk, half)], ewbuf)
            pltpu.async_copy(y_hbm.at[rowbuf.at[0]], rows0, sem0)
            lax.fori_loop(0, hpairs, pair, 0)

        plsc.subcore_barrier()

        for i in range(n_dump):
            buf, sem = (rows0, sem0) if i % 2 == 0 else (rows1, sem1)
            if i >= 2:
                pltpu.make_async_copy(dummy, buf, sem).wait()
            base = s * seg + i * _CH
            pltpu.sync_copy(acc_sh.at[pl.ds(base, _CH)], buf)
            pltpu.async_copy(buf, out_hbm.at[pl.ds(c * n_pad + base, _CH)],
                             sem)
        for i in range(max(0, n_dump - 2), n_dump):
            buf, sem = (rows0, sem0) if i % 2 == 0 else (rows1, sem1)
            pltpu.make_async_copy(dummy, buf, sem).wait()

    run = pl.kernel(
        body,
        out_type=jax.ShapeDtypeStruct((_NC * n_pad, d), jnp.float32),
        mesh=mesh,
        scratch_types=[
            pltpu.VMEM((half, _CH), jnp.int32),
            pltpu.VMEM((half, _CH), jnp.int32),
            pltpu.VMEM((half, _CH), jnp.float32),
            pltpu.VMEM((_CH, d), jnp.float32),
            pltpu.VMEM((_CH, d), jnp.float32),
            pltpu.VMEM_SHARED((n_pad, d), jnp.float32),
            pltpu.SemaphoreType.DMA,
            pltpu.SemaphoreType.DMA,
        ],
    )
    return run(row, col, ew, y)


def _tc_norm_y(d0, d1, x, W, rb):
    n, d = x.shape

    def body(d0_ref, d1_ref, x_ref, w_ref, dis_ref, y_ref):
        deg = 1.0 + d0_ref[...] + d1_ref[...]
        dis = lax.rsqrt(deg)
        xw = lax.dot_general(x_ref[...], w_ref[...],
                             (((1,), (1,)), ((), ())),
                             preferred_element_type=jnp.float32)
        dis_ref[...] = dis
        y_ref[...] = dis * xw

    return pl.pallas_call(
        body,
        grid=(n // rb,),
        in_specs=[
            pl.BlockSpec((rb, 1), lambda i: (i, 0)),
            pl.BlockSpec((rb, 1), lambda i: (i, 0)),
            pl.BlockSpec((rb, d), lambda i: (i, 0)),
            pl.BlockSpec((d, d), lambda i: (0, 0)),
        ],
        out_specs=[
            pl.BlockSpec((rb, 1), lambda i: (i, 0)),
            pl.BlockSpec((rb, d), lambda i: (i, 0)),
        ],
        out_shape=[
            jax.ShapeDtypeStruct((n, 1), jnp.float32),
            jax.ShapeDtypeStruct((n, d), jnp.float32),
        ],
    )(d0, d1, x, W)


def _tc_combine(dis, y, p0, p1, rb):
    n, d = y.shape

    def body(dis_ref, y_ref, p0_ref, p1_ref, o_ref):
        o_ref[...] = dis_ref[...] * (p0_ref[...] + p1_ref[...] + y_ref[...])

    return pl.pallas_call(
        body,
        grid=(n // rb,),
        in_specs=[
            pl.BlockSpec((rb, 1), lambda i: (i, 0)),
            pl.BlockSpec((rb, d), lambda i: (i, 0)),
            pl.BlockSpec((rb, d), lambda i: (i, 0)),
            pl.BlockSpec((rb, d), lambda i: (i, 0)),
        ],
        out_specs=pl.BlockSpec((rb, d), lambda i: (i, 0)),
        out_shape=jax.ShapeDtypeStruct((n, d), jnp.float32),
    )(dis, y, p0, p1)


@jax.jit
def kernel(x, edge_index, edge_weight, W):
    n, d = x.shape
    e = edge_weight.shape[0]
    ntile = _NC * _NS

    quantum = ntile * _CH * 4
    ep = ((e + quantum - 1) // quantum) * quantum
    n_chunks = ep // (ntile * _CH)
    pad = ep - e
    row = jnp.concatenate([edge_index[0], jnp.zeros((pad,), jnp.int32)])
    col = jnp.concatenate([edge_index[1], jnp.zeros((pad,), jnp.int32)])
    ewp = jnp.concatenate([edge_weight, jnp.zeros((pad,), jnp.float32)])
    row2 = row.reshape(ntile * n_chunks, _CH)
    col2 = col.reshape(ntile * n_chunks, _CH)
    ew2 = ewp.reshape(ntile * n_chunks, _CH)

    n_pad = ((n + _NS * _CH - 1) // (_NS * _CH)) * (_NS * _CH)

    dp = _sc_degree(col2, ew2, n_pad, n_chunks).reshape(_NC, n_pad)
    d0 = dp[0, :n, None]
    d1 = dp[1, :n, None]

    rb = 1000 if n % 1000 == 0 else 8
    dis, y = _tc_norm_y(d0, d1, x, W, rb)

    partials = _sc_message_scatter(row2, col2, ew2, y, n_pad,
                                   n_chunks, d).reshape(_NC, n_pad, d)
    p0 = partials[0, :n]
    p1 = partials[1, :n]

    return _tc_combine(dis, y, p0, p1, rb)
